# baseline (speedup 1.0000x reference)
import functools

import jax
import jax.numpy as jnp
from jax import lax
from jax.experimental import pallas as pl
from jax.experimental.pallas import tpu as pltpu
from jax.experimental.pallas import tpu_sc as plsc

VOCAB = 100000
HIDDEN = 128
OUT = 1000
B = 4096
L = 50

NC = 2
NS = 16
LANES = 16
NW = NC * NS
B_PER_W = B // NW
NVEC = HIDDEN // LANES
NBUF = 8


def _bag_body(idx_hbm, table_hbm, out_hbm, idx_v, rows_v, pool_v, *sems):
    wid = lax.axis_index("s") * NC + lax.axis_index("c")

    pltpu.sync_copy(idx_hbm.at[pl.ds(wid * B_PER_W, B_PER_W)], idx_v)

    def start_gather(s, buf):
        pltpu.async_copy(table_hbm.at[idx_v.at[s]], rows_v.at[buf], sems[buf])

    def wait_gather(s, buf):
        pltpu.make_async_copy(table_hbm.at[idx_v.at[s]], rows_v.at[buf],
                              sems[buf]).wait()

    for b in range(NBUF - 1):
        start_gather(b, b)

    def accum_sample(s, buf):
        def row_body(l, accs):
            new = list(accs)
            for u in range(2):
                r = 2 * l + u
                for h in range(NVEC):
                    new[h] = new[h] + rows_v[buf, r, pl.ds(h * LANES, LANES)]
            return tuple(new)

        zeros = tuple(jnp.zeros((LANES,), jnp.float32) for _ in range(NVEC))
        accs = lax.fori_loop(0, L // 2, row_body, zeros)
        for h in range(NVEC):
            pool_v[s, pl.ds(h * LANES, LANES)] = accs[h]

    def outer(jj, carry):
        for b in range(NBUF):
            s = jj * NBUF + b
            wait_gather(s, b)

            @pl.when(s + NBUF - 1 < B_PER_W)
            def _():
                start_gather(s + NBUF - 1, (b + NBUF - 1) % NBUF)

            accum_sample(s, b)
        return carry

    lax.fori_loop(0, B_PER_W // NBUF, outer, 0)

    pltpu.sync_copy(pool_v, out_hbm.at[pl.ds(wid * B_PER_W, B_PER_W)])


_bag = functools.partial(
    pl.kernel,
    out_type=jax.ShapeDtypeStruct((B, HIDDEN), jnp.float32),
    mesh=plsc.VectorSubcoreMesh(core_axis_name="c", subcore_axis_name="s"),
    scratch_types=[
        pltpu.VMEM((B_PER_W, L), jnp.int32),
        pltpu.VMEM((NBUF, L, HIDDEN), jnp.float32),
        pltpu.VMEM((B_PER_W, HIDDEN), jnp.float32),
    ] + [pltpu.SemaphoreType.DMA] * NBUF,
)(_bag_body)


def _mm_body(wt_ref, x_ref, b_ref, o_ref):
    o_ref[...] = (
        lax.dot_general(wt_ref[...], x_ref[...],
                        dimension_numbers=(((1,), (1,)), ((), ())),
                        preferred_element_type=jnp.float32)
        * (1.0 / L)
        + b_ref[...])


def _matmul_t(pooled, fc_wt, fc_b2):
    bn = 40
    return pl.pallas_call(
        _mm_body,
        grid=(OUT // bn,),
        in_specs=[
            pl.BlockSpec((bn, HIDDEN), lambda i: (i, 0)),
            pl.BlockSpec((B, HIDDEN), lambda i: (0, 0)),
            pl.BlockSpec((bn, 1), lambda i: (i, 0)),
        ],
        out_specs=pl.BlockSpec((bn, B), lambda i: (i, 0)),
        out_shape=jax.ShapeDtypeStruct((OUT, B), jnp.float32),
    )(fc_wt, pooled, fc_b2)


def kernel(x, emb_table, fc_w, fc_b):
    pooled = _bag(x.astype(jnp.int32), emb_table)
    out_t = _matmul_t(pooled, fc_w.T, fc_b.reshape(OUT, 1))
    return out_t.T


# speedup vs baseline: 1.1799x; 1.1799x over previous
import functools

import jax
import jax.numpy as jnp
from jax import lax
from jax.experimental import pallas as pl
from jax.experimental.pallas import tpu as pltpu
from jax.experimental.pallas import tpu_sc as plsc

VOCAB = 100000
HIDDEN = 128
OUT = 1000
B = 4096
L = 50

NC = 2
NS = 16
LANES = 16
NW = NC * NS
B_PER_W = B // NW
NVEC = HIDDEN // LANES
NBUF = 8


def _bag_body(idx_hbm, table_hbm, out_hbm, idx_v, rows_v, pool_v, *sems):
    wid = lax.axis_index("s") * NC + lax.axis_index("c")

    pltpu.sync_copy(idx_hbm.at[pl.ds(wid * B_PER_W, B_PER_W)], idx_v)

    def start_gather(s, buf):
        pltpu.async_copy(table_hbm.at[idx_v.at[s]], rows_v.at[buf], sems[buf])

    def wait_gather(s, buf):
        pltpu.make_async_copy(table_hbm.at[idx_v.at[s]], rows_v.at[buf],
                              sems[buf]).wait()

    for b in range(NBUF - 1):
        start_gather(b, b)

    def accum_sample(s, buf):
        def row_body(l, accs):
            new = list(accs)
            for u in range(2):
                r = 2 * l + u
                for h in range(NVEC):
                    new[h] = new[h] + rows_v[buf, r, pl.ds(h * LANES, LANES)]
            return tuple(new)

        zeros = tuple(jnp.zeros((LANES,), jnp.float32) for _ in range(NVEC))
        accs = lax.fori_loop(0, L // 2, row_body, zeros)
        for h in range(NVEC):
            pool_v[s, pl.ds(h * LANES, LANES)] = accs[h]

    def outer(jj, carry):
        for b in range(NBUF):
            s = jj * NBUF + b
            wait_gather(s, b)

            @pl.when(s + NBUF - 1 < B_PER_W)
            def _():
                start_gather(s + NBUF - 1, (b + NBUF - 1) % NBUF)

            accum_sample(s, b)
        return carry

    lax.fori_loop(0, B_PER_W // NBUF, outer, 0)

    pltpu.sync_copy(pool_v, out_hbm.at[pl.ds(wid * B_PER_W, B_PER_W)])


_bag = functools.partial(
    pl.kernel,
    out_type=jax.ShapeDtypeStruct((B, HIDDEN), jnp.float32),
    mesh=plsc.VectorSubcoreMesh(core_axis_name="c", subcore_axis_name="s"),
    scratch_types=[
        pltpu.VMEM((B_PER_W, L), jnp.int32),
        pltpu.VMEM((NBUF, L, HIDDEN), jnp.float32),
        pltpu.VMEM((B_PER_W, HIDDEN), jnp.float32),
    ] + [pltpu.SemaphoreType.DMA] * NBUF,
)(_bag_body)


def _mm_body(wt_ref, x_ref, b_ref, o_ref):
    o_ref[...] = (
        lax.dot_general(wt_ref[...], x_ref[...],
                        dimension_numbers=(((1,), (1,)), ((), ())),
                        preferred_element_type=jnp.float32)
        * (1.0 / L)
        + b_ref[...])


def _matmul_t(pooled, fc_wt, fc_b2):
    bn = 200
    return pl.pallas_call(
        _mm_body,
        grid=(OUT // bn,),
        in_specs=[
            pl.BlockSpec((bn, HIDDEN), lambda i: (i, 0)),
            pl.BlockSpec((B, HIDDEN), lambda i: (0, 0)),
            pl.BlockSpec((bn, 1), lambda i: (i, 0)),
        ],
        out_specs=pl.BlockSpec((bn, B), lambda i: (i, 0)),
        out_shape=jax.ShapeDtypeStruct((OUT, B), jnp.float32),
    )(fc_wt, pooled, fc_b2)


def kernel(x, emb_table, fc_w, fc_b):
    pooled = _bag(x.astype(jnp.int32), emb_table)
    out_t = _matmul_t(pooled, fc_w.T, fc_b.reshape(OUT, 1))
    return out_t.T
